# baseline (speedup 1.0000x reference)
import numpy as np
import jax
import jax.numpy as jnp
from jax.experimental import pallas as pl

NUM_K = 8192
NUM_Q = 512
DIM_ATT = 64
TOP_K = 64
D = 1024
B = 8
SCORE_MIN = 1e-25
SCORE_MAX = 1e+25

QT = 128
QG = NUM_Q // QT
KC = 8
K_BLK = NUM_K // KC

_LO_BITS = 0
_HI_BITS = int(np.float32(2.0).view(np.int32))


def _topk_body(key_ref, query_ref, pmap_ref):
    k = key_ref[:]
    q = query_ref[:]
    kn = k / jnp.maximum(jnp.sqrt(jnp.sum(k * k, axis=0, keepdims=True)), 1e-12)
    qn = q / jnp.maximum(jnp.sqrt(jnp.sum(q * q, axis=0, keepdims=True)), 1e-12)
    s = jax.lax.dot_general(
        qn.astype(jnp.bfloat16), kn.astype(jnp.bfloat16),
        (((0,), (0,)), ((), ())),
        preferred_element_type=jnp.float32)
    s = jnp.clip(s, SCORE_MIN, SCORE_MAX)

    def count_ge(t_bits):
        tf = jax.lax.bitcast_convert_type(t_bits, jnp.float32)
        return jnp.sum((s >= tf).astype(jnp.int32), axis=1, keepdims=True)

    lo = jnp.full((QT, 1), _LO_BITS, jnp.int32)
    hi = jnp.full((QT, 1), _HI_BITS, jnp.int32)
    c_lo = jnp.full((QT, 1), NUM_K, jnp.int32)

    def _plog(c):
        return jax.lax.bitcast_convert_type(
            (c.astype(jnp.float32) + 0.5), jnp.int32)

    _PLOG_TARGET = int(np.float32(float(TOP_K) + 0.5).view(np.int32))

    def _live(lo, hi, c_lo, c_hi):
        return (hi - lo > 1) & (c_lo != TOP_K) & (c_hi != TOP_K - 1)

    _SEED0 = int(np.float32(0.375).view(np.int32))
    _SEED1 = int(np.float32(0.25).view(np.int32))

    def w_cond(carry):
        it, lo, hi, c_lo, c_hi = carry
        return (it < 72) & jnp.any(_live(lo, hi, c_lo, c_hi))

    def w_body(carry):
        it, lo, hi, c_lo, c_hi = carry
        d = hi - lo
        num = (_plog(c_lo) - _PLOG_TARGET).astype(jnp.float32)
        den = (_plog(c_lo) - _plog(c_hi)).astype(jnp.float32)
        m_int = lo + (d.astype(jnp.float32) * (num / den)).astype(jnp.int32)
        m = (jnp.clip(m_int, lo + 1, hi - 1) + (lo + (d >> 1))) >> 1
        m = jnp.where(it == 0, _SEED0, jnp.where(it == 1, _SEED1, m))
        live = _live(lo, hi, c_lo, c_hi)
        m = jnp.where(live, jnp.clip(m, lo + 1, hi - 1), lo)
        c = count_ge(m)
        q = c >= TOP_K
        new_lo = jnp.where(live & q, m, lo)
        new_clo = jnp.where(live & q, c, c_lo)
        new_hi = jnp.where(live & ~q, m, hi)
        new_chi = jnp.where(live & ~q, c, c_hi)
        return it + 1, new_lo, new_hi, new_clo, new_chi

    _, lo, hi, c_lo, c_hi = jax.lax.while_loop(
        w_cond, w_body,
        (jnp.int32(0), lo, hi, c_lo, jnp.zeros((QT, 1), jnp.int32)))

    tf_lo = jax.lax.bitcast_convert_type(lo, jnp.float32)
    tf_hi = jax.lax.bitcast_convert_type(hi, jnp.float32)
    mmin = jnp.min(jnp.where(s >= tf_lo, s, jnp.inf), axis=1, keepdims=True)
    mmax = jnp.max(jnp.where(s < tf_hi, s, -jnp.inf), axis=1, keepdims=True)
    tf = jnp.where(c_lo == TOP_K, mmin,
                   jnp.where(c_hi == TOP_K - 1, mmax, tf_lo))

    c_gt = jnp.sum((s > tf).astype(jnp.int32), axis=1, keepdims=True)
    c_geq = jnp.sum((s >= tf).astype(jnp.int32), axis=1, keepdims=True)
    r = TOP_K - c_gt

    iota = jax.lax.broadcasted_iota(jnp.int32, (QT, NUM_K), 1)

    def tie_search():
        eq = (s == tf)

        def jstep(_, carry):
            loj, hij = carry
            mj = (loj + hij + 1) >> 1
            cnt = jnp.sum((eq & (iota <= mj)).astype(jnp.int32),
                          axis=1, keepdims=True)
            take = cnt >= r
            return jnp.where(take, loj, mj), jnp.where(take, mj, hij)

        loj = jnp.full((QT, 1), -1, jnp.int32)
        hij = jnp.full((QT, 1), NUM_K - 1, jnp.int32)
        _, hij = jax.lax.fori_loop(0, 13, jstep, (loj, hij))
        return hij

    jstar = jax.lax.cond(jnp.max(c_geq) > TOP_K, tie_search,
                         lambda: jnp.full((QT, 1), NUM_K, jnp.int32))

    selected = (s > tf) | ((s == tf) & (iota <= jstar))
    praw = jnp.where(selected, s, 0.0)
    ssum = jnp.sum(praw, axis=1, keepdims=True)
    pmap_ref[:] = praw / jnp.maximum(ssum, 1e-12)


def _topk_pmap(key, query):
    return pl.pallas_call(
        _topk_body,
        grid=(QG,),
        in_specs=[
            pl.BlockSpec((DIM_ATT, NUM_K), lambda g: (0, 0)),
            pl.BlockSpec((DIM_ATT, QT), lambda g: (0, g)),
        ],
        out_specs=pl.BlockSpec((QT, NUM_K), lambda g: (g, 0)),
        out_shape=jax.ShapeDtypeStruct((NUM_Q, NUM_K), jnp.float32),
    )(key, query)


def _pool_body(pmap_ref, x_ref, out_ref):
    kc = pl.program_id(0)
    b = pl.program_id(1)
    acc = jax.lax.dot_general(
        pmap_ref[:], x_ref[0], (((1,), (0,)), ((), ())),
        preferred_element_type=jnp.float32)

    @pl.when(kc == 0)
    def _():
        out_ref[b] = acc

    @pl.when(kc != 0)
    def _():
        out_ref[b] += acc


def _pool(pmap, x):
    return pl.pallas_call(
        _pool_body,
        grid=(KC, B),
        in_specs=[
            pl.BlockSpec((NUM_Q, K_BLK), lambda kc, b: (0, kc)),
            pl.BlockSpec((1, K_BLK, D), lambda kc, b: (b, kc, 0)),
        ],
        out_specs=pl.BlockSpec((B, NUM_Q, D), lambda kc, b: (0, 0, 0)),
        out_shape=jax.ShapeDtypeStruct((B, NUM_Q, D), jnp.float32),
    )(pmap, x)


def kernel(x, key, query):
    pmap = _topk_pmap(key, query)
    return _pool(pmap, x)


# speedup vs baseline: 1.0012x; 1.0012x over previous
import numpy as np
import jax
import jax.numpy as jnp
from jax.experimental import pallas as pl

NUM_K = 8192
NUM_Q = 512
DIM_ATT = 64
TOP_K = 64
D = 1024
B = 8
SCORE_MIN = 1e-25
SCORE_MAX = 1e+25

QT = 128
QG = NUM_Q // QT
KC = 8
K_BLK = NUM_K // KC

_LO_BITS = 0
_HI_BITS = int(np.float32(2.0).view(np.int32))


def _topk_body(key_ref, query_ref, pmap_ref):
    k = key_ref[:]
    q = query_ref[:]
    kn = k / jnp.maximum(jnp.sqrt(jnp.sum(k * k, axis=0, keepdims=True)), 1e-12)
    qn = q / jnp.maximum(jnp.sqrt(jnp.sum(q * q, axis=0, keepdims=True)), 1e-12)
    s = jax.lax.dot_general(
        qn.astype(jnp.bfloat16), kn.astype(jnp.bfloat16),
        (((0,), (0,)), ((), ())),
        preferred_element_type=jnp.float32)
    s = jnp.clip(s, SCORE_MIN, SCORE_MAX)

    def count_ge(t_bits):
        tf = jax.lax.bitcast_convert_type(t_bits, jnp.float32)
        return jnp.sum((s >= tf).astype(jnp.int32), axis=1, keepdims=True)

    lo = jnp.full((QT, 1), _LO_BITS, jnp.int32)
    hi = jnp.full((QT, 1), _HI_BITS, jnp.int32)
    c_lo = jnp.full((QT, 1), NUM_K, jnp.int32)

    def _plog(c):
        return jax.lax.bitcast_convert_type(
            (c.astype(jnp.float32) + 0.5), jnp.int32)

    _PLOG_TARGET = int(np.float32(float(TOP_K) + 0.5).view(np.int32))

    def _live(lo, hi, c_lo, c_hi):
        return (hi - lo > 1) & (c_lo != TOP_K) & (c_hi != TOP_K - 1)

    _SEED0 = int(np.float32(0.375).view(np.int32))
    _SEED1 = int(np.float32(0.25).view(np.int32))

    def w_cond(carry):
        it, lo, hi, c_lo, c_hi = carry
        return (it < 80) & jnp.any(_live(lo, hi, c_lo, c_hi))

    def w_body(carry):
        it, lo, hi, c_lo, c_hi = carry
        d = hi - lo
        num = (_plog(c_lo) - _PLOG_TARGET).astype(jnp.float32)
        den = (_plog(c_lo) - _plog(c_hi)).astype(jnp.float32)
        m_int = lo + (d.astype(jnp.float32) * (num / den)).astype(jnp.int32)
        m = (jnp.clip(m_int, lo + 1, hi - 1) + (lo + (d >> 1))) >> 1
        m = jnp.where(it == 0, _SEED0, jnp.where(it == 1, _SEED1, m))
        live = _live(lo, hi, c_lo, c_hi)
        m = jnp.where(live, jnp.clip(m, lo + 1, hi - 1), lo)
        c = count_ge(m)
        q = c >= TOP_K
        new_lo = jnp.where(live & q, m, lo)
        new_clo = jnp.where(live & q, c, c_lo)
        new_hi = jnp.where(live & ~q, m, hi)
        new_chi = jnp.where(live & ~q, c, c_hi)
        return it + 1, new_lo, new_hi, new_clo, new_chi

    _, lo, hi, c_lo, c_hi = jax.lax.while_loop(
        w_cond, w_body,
        (jnp.int32(0), lo, hi, c_lo, jnp.zeros((QT, 1), jnp.int32)))

    tf_lo = jax.lax.bitcast_convert_type(lo, jnp.float32)
    tf_hi = jax.lax.bitcast_convert_type(hi, jnp.float32)
    mmin = jnp.min(jnp.where(s >= tf_lo, s, jnp.inf), axis=1, keepdims=True)
    mmax = jnp.max(jnp.where(s < tf_hi, s, -jnp.inf), axis=1, keepdims=True)
    tf = jnp.where(c_lo == TOP_K, mmin,
                   jnp.where(c_hi == TOP_K - 1, mmax, tf_lo))

    c_gt = jnp.sum((s > tf).astype(jnp.int32), axis=1, keepdims=True)
    c_geq = jnp.sum((s >= tf).astype(jnp.int32), axis=1, keepdims=True)
    r = TOP_K - c_gt

    iota = jax.lax.broadcasted_iota(jnp.int32, (QT, NUM_K), 1)

    def tie_search():
        eq = (s == tf)

        def jstep(_, carry):
            loj, hij = carry
            mj = (loj + hij + 1) >> 1
            cnt = jnp.sum((eq & (iota <= mj)).astype(jnp.int32),
                          axis=1, keepdims=True)
            take = cnt >= r
            return jnp.where(take, loj, mj), jnp.where(take, mj, hij)

        loj = jnp.full((QT, 1), -1, jnp.int32)
        hij = jnp.full((QT, 1), NUM_K - 1, jnp.int32)
        _, hij = jax.lax.fori_loop(0, 13, jstep, (loj, hij))
        return hij

    jstar = jax.lax.cond(jnp.max(c_geq) > TOP_K, tie_search,
                         lambda: jnp.full((QT, 1), NUM_K, jnp.int32))

    selected = (s > tf) | ((s == tf) & (iota <= jstar))
    praw = jnp.where(selected, s, 0.0)
    ssum = jnp.sum(praw, axis=1, keepdims=True)
    pmap_ref[:] = praw / jnp.maximum(ssum, 1e-12)


def _topk_pmap(key, query):
    return pl.pallas_call(
        _topk_body,
        grid=(QG,),
        in_specs=[
            pl.BlockSpec((DIM_ATT, NUM_K), lambda g: (0, 0)),
            pl.BlockSpec((DIM_ATT, QT), lambda g: (0, g)),
        ],
        out_specs=pl.BlockSpec((QT, NUM_K), lambda g: (g, 0)),
        out_shape=jax.ShapeDtypeStruct((NUM_Q, NUM_K), jnp.float32),
    )(key, query)


def _pool_body(pmap_ref, x_ref, out_ref):
    kc = pl.program_id(0)
    b = pl.program_id(1)
    acc = jax.lax.dot_general(
        pmap_ref[:], x_ref[0], (((1,), (0,)), ((), ())),
        preferred_element_type=jnp.float32)

    @pl.when(kc == 0)
    def _():
        out_ref[b] = acc

    @pl.when(kc != 0)
    def _():
        out_ref[b] += acc


def _pool(pmap, x):
    return pl.pallas_call(
        _pool_body,
        grid=(KC, B),
        in_specs=[
            pl.BlockSpec((NUM_Q, K_BLK), lambda kc, b: (0, kc)),
            pl.BlockSpec((1, K_BLK, D), lambda kc, b: (b, kc, 0)),
        ],
        out_specs=pl.BlockSpec((B, NUM_Q, D), lambda kc, b: (0, 0, 0)),
        out_shape=jax.ShapeDtypeStruct((B, NUM_Q, D), jnp.float32),
    )(pmap, x)


def kernel(x, key, query):
    pmap = _topk_pmap(key, query)
    return _pool(pmap, x)
